# baseline (speedup 1.0000x reference)
import functools

import jax
import jax.numpy as jnp
from jax import lax
from jax.experimental import pallas as pl
from jax.experimental.pallas import tpu as pltpu
from jax.experimental.pallas import tpu_sc as plsc

D = 1024
H = 2048
E = 64
T = 4096

BLOCK = 128
NB = 2 * T // BLOCK + E
P_MAX = NB * BLOCK

NC, NS = 2, 16
NW = NC * NS

RT = 256


NT = T // RT


def _router_body(x_ref, wg_ref, ei_ref, ws_ref, hist_ref):
    s = lax.dot_general(x_ref[...], wg_ref[...], (((1,), (1,)), ((), ())),
                        preferred_element_type=jnp.float32)
    ecol = lax.broadcasted_iota(jnp.int32, (RT, E), 1)
    m1 = jnp.max(s, axis=1, keepdims=True)
    i1 = jnp.min(jnp.where(s == m1, ecol, E), axis=1, keepdims=True)
    s2 = jnp.where(ecol == i1, -1e30, s)
    m2 = jnp.max(s2, axis=1, keepdims=True)
    i2 = jnp.min(jnp.where(s2 == m2, ecol, E), axis=1, keepdims=True)
    w1 = 1.0 / (1.0 + jnp.exp(m2 - m1))
    ei_ref[...] = jnp.concatenate([i1, i2], axis=1)
    ws_ref[...] = jnp.concatenate([w1, 1.0 - w1], axis=1)
    cnt = jnp.sum(((ecol == i1) | (ecol == i2)).astype(jnp.float32), axis=0,
                  keepdims=True)
    hist_ref[...] = jnp.concatenate(
        [cnt, jnp.zeros((7, E), jnp.float32)], axis=0)[None]


def _router(x2d, Wg):
    return pl.pallas_call(
        _router_body,
        grid=(NT,),
        in_specs=[
            pl.BlockSpec((RT, D), lambda i: (i, 0)),
            pl.BlockSpec((E, D), lambda i: (0, 0)),
        ],
        out_specs=[
            pl.BlockSpec((RT, 2), lambda i: (i, 0)),
            pl.BlockSpec((RT, 2), lambda i: (i, 0)),
            pl.BlockSpec((1, 8, E), lambda i: (i, 0, 0)),
        ],
        out_shape=[
            jax.ShapeDtypeStruct((T, 2), jnp.int32),
            jax.ShapeDtypeStruct((T, 2), jnp.float32),
            jax.ShapeDtypeStruct((NT, 8, E), jnp.float32),
        ],
    )(x2d, Wg)


def _rank_body(ei_ref, cb_ref, qp_ref):
    e0 = ei_ref[:, 0:1]
    e1 = ei_ref[:, 1:2]
    ecol = lax.broadcasted_iota(jnp.int32, (RT, E), 1)
    oh0 = (ecol == e0).astype(jnp.float32)
    oh1 = (ecol == e1).astype(jnp.float32)
    rowcnt = oh0 + oh1
    ri = lax.broadcasted_iota(jnp.int32, (RT, RT), 0)
    ci = lax.broadcasted_iota(jnp.int32, (RT, RT), 1)
    ltri = (ci < ri).astype(jnp.float32)
    prior = lax.dot_general(ltri, rowcnt, (((1,), (0,)), ((), ())),
                            preferred_element_type=jnp.float32)
    base = cb_ref[0, 0:1, :] + prior
    q0 = jnp.sum(base * oh0, axis=1, keepdims=True)
    q1 = jnp.sum(base * oh1, axis=1, keepdims=True) + (e0 == e1)
    qp_ref[...] = jnp.concatenate([q0, q1], axis=1).astype(jnp.int32)


def _rank(ei, cbase):
    return pl.pallas_call(
        _rank_body,
        grid=(NT,),
        in_specs=[
            pl.BlockSpec((RT, 2), lambda i: (i, 0)),
            pl.BlockSpec((1, 8, E), lambda i: (i, 0, 0)),
        ],
        out_specs=pl.BlockSpec((RT, 2), lambda i: (i, 0)),
        out_shape=jax.ShapeDtypeStruct((T, 2), jnp.int32),
    )(ei, cbase)


def _bookkeep(ei, ws, hist3):
    i32 = jnp.int32
    hist = hist3[:, 0, :]
    counts = jnp.sum(hist, axis=0).astype(i32)
    tilebase = jnp.cumsum(hist, axis=0) - hist
    nblk = (counts + BLOCK - 1) // BLOCK
    pstart = (jnp.cumsum(nblk) - nblk) * BLOCK
    cbase = pstart.astype(jnp.float32)[None, :] + tilebase
    cbase3 = jnp.broadcast_to(cbase[:, None, :], (NT, 8, E))
    qp = _rank(ei, cbase3)
    qf = qp.reshape(-1)
    tokf = jnp.arange(2 * T, dtype=i32) // 2
    src_idx = jnp.zeros((P_MAX,), i32).at[qf].set(tokf)
    wpad = jnp.zeros((P_MAX,), jnp.float32).at[qf].set(ws.reshape(-1))
    bids = jnp.arange(NB, dtype=i32)
    be = jnp.clip(
        jnp.searchsorted(pstart, bids * BLOCK, side="right").astype(i32) - 1,
        0, E - 1)
    bvalid = (bids * BLOCK < jnp.sum(nblk) * BLOCK).astype(i32)
    return src_idx, wpad, qp[:, 0], qp[:, 1], be, bvalid


G = 4
PG = P_MAX // G
NBG = NB // G

_D_CH = 32
_D_PER_W = PG // NW
_D_NCH = _D_PER_W // _D_CH


def _dispatch_body(x_hbm, idx_hbm, xs_hbm, idx_v, r0, r1, s0, s1):
    wid = lax.axis_index("s") * NC + lax.axis_index("c")
    pltpu.sync_copy(idx_hbm.at[wid], idx_v)
    base = wid * _D_PER_W
    bufs, sems = (r0, r1), (s0, s1)

    def start(c, b):
        pltpu.async_copy(x_hbm.at[idx_v.at[c]], bufs[b], sems[b])

    def wait(b):
        pltpu.make_async_copy(x_hbm.at[idx_v.at[0]], bufs[b], sems[b]).wait()

    start(0, 0)
    for c in range(_D_NCH):
        b = c % 2
        wait(b)
        if c + 1 < _D_NCH:
            start(c + 1, 1 - b)
        pltpu.sync_copy(bufs[b], xs_hbm.at[pl.ds(base + c * _D_CH, _D_CH)])


def _dispatch(x2d, src_idx_g):
    mesh = plsc.VectorSubcoreMesh(core_axis_name="c", subcore_axis_name="s")
    f = pl.kernel(
        _dispatch_body,
        out_type=jax.ShapeDtypeStruct((PG, D), jnp.float32),
        mesh=mesh,
        scratch_types=[
            pltpu.VMEM((_D_NCH, _D_CH), jnp.int32),
            pltpu.VMEM((_D_CH, D), jnp.float32),
            pltpu.VMEM((_D_CH, D), jnp.float32),
            pltpu.SemaphoreType.DMA,
            pltpu.SemaphoreType.DMA,
        ],
    )
    return f(x2d, src_idx_g.reshape(NW, _D_NCH, _D_CH))


def _ffn_body(be_ref, bv_ref, xs_ref, w1_ref, w3_ref, w2_ref, wp_ref, *rest):
    ys_ref = rest[-1]
    i = pl.program_id(0)

    @pl.when(bv_ref[i] == 1)
    def _():
        xt = xs_ref[...]
        a = lax.dot_general(xt, w1_ref[0], (((1,), (1,)), ((), ())),
                            preferred_element_type=jnp.float32)
        g = lax.dot_general(xt, w3_ref[0], (((1,), (1,)), ((), ())),
                            preferred_element_type=jnp.float32)
        h = (a / (1.0 + jnp.exp(-a))) * g
        o = lax.dot_general(h, w2_ref[0], (((1,), (1,)), ((), ())),
                            preferred_element_type=jnp.float32)
        ys_ref[...] = o * wp_ref[0]


def _ffn_group(g, xs_g, wpad_g, be_g, bv_g, W1, W3, W2, ys_prev):
    in_specs = [
        pl.BlockSpec((BLOCK, D), lambda i, be, bv: (i, 0)),
        pl.BlockSpec((1, H, D), lambda i, be, bv: (be[i], 0, 0)),
        pl.BlockSpec((1, H, D), lambda i, be, bv: (be[i], 0, 0)),
        pl.BlockSpec((1, D, H), lambda i, be, bv: (be[i], 0, 0)),
        pl.BlockSpec((1, BLOCK, 1), lambda i, be, bv: (i, 0, 0)),
    ]
    args = [be_g, bv_g, xs_g, W1, W3, W2, wpad_g.reshape(NBG, BLOCK, 1)]
    aliases = {}
    if ys_prev is not None:
        in_specs.append(pl.BlockSpec(memory_space=pl.ANY))
        args.append(ys_prev)
        aliases = {7: 0}
    grid_spec = pltpu.PrefetchScalarGridSpec(
        num_scalar_prefetch=2,
        grid=(NBG,),
        in_specs=in_specs,
        out_specs=pl.BlockSpec((BLOCK, D),
                               lambda i, be, bv, g=g: (i + g * NBG, 0)),
    )
    return pl.pallas_call(
        _ffn_body,
        grid_spec=grid_spec,
        out_shape=jax.ShapeDtypeStruct((P_MAX, D), jnp.float32),
        input_output_aliases=aliases,
        compiler_params=pltpu.CompilerParams(
            dimension_semantics=("arbitrary",),
            vmem_limit_bytes=128 * 1024 * 1024,
        ),
    )(*args)


_C_CH = 32
_C_PER_W = T // NW
_C_NCH = _C_PER_W // _C_CH


def _combine_body(ys_hbm, q1_hbm, q2_hbm, out_hbm, q1_v, q2_v, bufa, bufb, sem):
    wid = lax.axis_index("s") * NC + lax.axis_index("c")
    pltpu.sync_copy(q1_hbm.at[wid], q1_v)
    pltpu.sync_copy(q2_hbm.at[wid], q2_v)
    base = wid * _C_PER_W
    for c in range(_C_NCH):
        pltpu.async_copy(ys_hbm.at[q1_v.at[c]], bufa, sem).wait()
        pltpu.async_copy(ys_hbm.at[q2_v.at[c]], bufb, sem).wait()

        def row(r, carry):
            def vec(v, carry2):
                sl = pl.ds(v * 16, 16)
                bufa[r, sl] = bufa[r, sl] + bufb[r, sl]
                return carry2
            return lax.fori_loop(0, D // 16, vec, carry)

        lax.fori_loop(0, _C_CH, row, 0)
        pltpu.sync_copy(bufa, out_hbm.at[pl.ds(base + c * _C_CH, _C_CH)])


def _combine(ys, q1, q2):
    mesh = plsc.VectorSubcoreMesh(core_axis_name="c", subcore_axis_name="s")
    f = pl.kernel(
        _combine_body,
        out_type=jax.ShapeDtypeStruct((T, D), jnp.float32),
        mesh=mesh,
        scratch_types=[
            pltpu.VMEM((_C_NCH, _C_CH), jnp.int32),
            pltpu.VMEM((_C_NCH, _C_CH), jnp.int32),
            pltpu.VMEM((_C_CH, D), jnp.float32),
            pltpu.VMEM((_C_CH, D), jnp.float32),
            pltpu.SemaphoreType.DMA,
        ],
    )
    return f(ys, q1.reshape(NW, _C_NCH, _C_CH), q2.reshape(NW, _C_NCH, _C_CH))


def kernel(x, Wg, W1, W3, W2):
    B, S, _ = x.shape
    x2d = x.reshape(T, D)
    ei, ws, hist3 = _router(x2d, Wg)
    src_idx, wpad, q1, q2, be, bvalid = _bookkeep(ei, ws, hist3)
    xs_groups = [_dispatch(x2d, src_idx[g * PG:(g + 1) * PG]) for g in range(G)]
    ys = None
    for g in range(G):
        ys = _ffn_group(
            g, xs_groups[g], wpad[g * PG:(g + 1) * PG],
            be[g * NBG:(g + 1) * NBG], bvalid[g * NBG:(g + 1) * NBG],
            W1, W3, W2, ys)
    out = _combine(ys, q1, q2)
    return out.reshape(B, S, D)


# speedup vs baseline: 1.0124x; 1.0124x over previous
import functools

import jax
import jax.numpy as jnp
from jax import lax
from jax.experimental import pallas as pl
from jax.experimental.pallas import tpu as pltpu
from jax.experimental.pallas import tpu_sc as plsc

D = 1024
H = 2048
E = 64
T = 4096

BLOCK = 128
NB = 2 * T // BLOCK + E
P_MAX = NB * BLOCK

NC, NS = 2, 16
NW = NC * NS

RT = 256


NT = T // RT


def _router_body(x_ref, wg_ref, ei_ref, ws_ref, hist_ref):
    s = lax.dot_general(x_ref[...], wg_ref[...], (((1,), (1,)), ((), ())),
                        preferred_element_type=jnp.float32)
    ecol = lax.broadcasted_iota(jnp.int32, (RT, E), 1)
    m1 = jnp.max(s, axis=1, keepdims=True)
    i1 = jnp.min(jnp.where(s == m1, ecol, E), axis=1, keepdims=True)
    s2 = jnp.where(ecol == i1, -1e30, s)
    m2 = jnp.max(s2, axis=1, keepdims=True)
    i2 = jnp.min(jnp.where(s2 == m2, ecol, E), axis=1, keepdims=True)
    w1 = 1.0 / (1.0 + jnp.exp(m2 - m1))
    ei_ref[...] = jnp.concatenate([i1, i2], axis=1)
    ws_ref[...] = jnp.concatenate([w1, 1.0 - w1], axis=1)
    cnt = jnp.sum(((ecol == i1) | (ecol == i2)).astype(jnp.float32), axis=0,
                  keepdims=True)
    hist_ref[...] = jnp.concatenate(
        [cnt, jnp.zeros((7, E), jnp.float32)], axis=0)[None]


def _router(x2d, Wg):
    return pl.pallas_call(
        _router_body,
        grid=(NT,),
        in_specs=[
            pl.BlockSpec((RT, D), lambda i: (i, 0)),
            pl.BlockSpec((E, D), lambda i: (0, 0)),
        ],
        out_specs=[
            pl.BlockSpec((RT, 2), lambda i: (i, 0)),
            pl.BlockSpec((RT, 2), lambda i: (i, 0)),
            pl.BlockSpec((1, 8, E), lambda i: (i, 0, 0)),
        ],
        out_shape=[
            jax.ShapeDtypeStruct((T, 2), jnp.int32),
            jax.ShapeDtypeStruct((T, 2), jnp.float32),
            jax.ShapeDtypeStruct((NT, 8, E), jnp.float32),
        ],
    )(x2d, Wg)


def _rank_body(ei_ref, cb_ref, qp_ref):
    e0 = ei_ref[:, 0:1]
    e1 = ei_ref[:, 1:2]
    ecol = lax.broadcasted_iota(jnp.int32, (RT, E), 1)
    oh0 = (ecol == e0).astype(jnp.float32)
    oh1 = (ecol == e1).astype(jnp.float32)
    rowcnt = oh0 + oh1
    ri = lax.broadcasted_iota(jnp.int32, (RT, RT), 0)
    ci = lax.broadcasted_iota(jnp.int32, (RT, RT), 1)
    ltri = (ci < ri).astype(jnp.float32)
    prior = lax.dot_general(ltri, rowcnt, (((1,), (0,)), ((), ())),
                            preferred_element_type=jnp.float32)
    base = cb_ref[0, 0:1, :] + prior
    q0 = jnp.sum(base * oh0, axis=1, keepdims=True)
    q1 = jnp.sum(base * oh1, axis=1, keepdims=True) + (e0 == e1)
    qp_ref[...] = jnp.concatenate([q0, q1], axis=1).astype(jnp.int32)


def _rank(ei, cbase):
    return pl.pallas_call(
        _rank_body,
        grid=(NT,),
        in_specs=[
            pl.BlockSpec((RT, 2), lambda i: (i, 0)),
            pl.BlockSpec((1, 8, E), lambda i: (i, 0, 0)),
        ],
        out_specs=pl.BlockSpec((RT, 2), lambda i: (i, 0)),
        out_shape=jax.ShapeDtypeStruct((T, 2), jnp.int32),
    )(ei, cbase)


def _bookkeep(ei, ws, hist3):
    i32 = jnp.int32
    hist = hist3[:, 0, :]
    counts = jnp.sum(hist, axis=0).astype(i32)
    tilebase = jnp.cumsum(hist, axis=0) - hist
    nblk = (counts + BLOCK - 1) // BLOCK
    pstart = (jnp.cumsum(nblk) - nblk) * BLOCK
    cbase = pstart.astype(jnp.float32)[None, :] + tilebase
    cbase3 = jnp.broadcast_to(cbase[:, None, :], (NT, 8, E))
    qp = _rank(ei, cbase3)
    qf = qp.reshape(-1)
    tokf = jnp.arange(2 * T, dtype=i32) // 2
    src_idx = jnp.zeros((P_MAX,), i32).at[qf].set(tokf)
    wpad = jnp.zeros((P_MAX,), jnp.float32).at[qf].set(ws.reshape(-1))
    bids = jnp.arange(NB, dtype=i32)
    be = jnp.clip(
        jnp.searchsorted(pstart, bids * BLOCK, side="right").astype(i32) - 1,
        0, E - 1)
    bvalid = (bids * BLOCK < jnp.sum(nblk) * BLOCK).astype(i32)
    return src_idx, wpad, qp[:, 0], qp[:, 1], be, bvalid


G = 1
PG = P_MAX // G
NBG = NB // G

_D_CH = 32
_D_PER_W = PG // NW
_D_NCH = _D_PER_W // _D_CH


_D_NBUF = 3


def _dispatch_body(x_hbm, idx_hbm, xs_hbm, idx_v, r0, r1, r2, s0, s1, s2):
    wid = lax.axis_index("s") * NC + lax.axis_index("c")
    pltpu.sync_copy(idx_hbm.at[wid], idx_v)
    base = wid * _D_PER_W
    bufs, sems = (r0, r1, r2), (s0, s1, s2)

    def start(c, b):
        pltpu.async_copy(x_hbm.at[idx_v.at[c]], bufs[b], sems[b])

    def wait(b):
        pltpu.make_async_copy(x_hbm.at[idx_v.at[0]], bufs[b], sems[b]).wait()

    start(0, 0)
    start(1, 1)
    for c in range(_D_NCH):
        b = c % _D_NBUF
        wait(b)
        if c + 2 < _D_NCH:
            start(c + 2, (c + 2) % _D_NBUF)
        pltpu.sync_copy(bufs[b], xs_hbm.at[pl.ds(base + c * _D_CH, _D_CH)])


def _dispatch(x2d, src_idx_g):
    mesh = plsc.VectorSubcoreMesh(core_axis_name="c", subcore_axis_name="s")
    f = pl.kernel(
        _dispatch_body,
        out_type=jax.ShapeDtypeStruct((PG, D), jnp.float32),
        mesh=mesh,
        scratch_types=[
            pltpu.VMEM((_D_NCH, _D_CH), jnp.int32),
            pltpu.VMEM((_D_CH, D), jnp.float32),
            pltpu.VMEM((_D_CH, D), jnp.float32),
            pltpu.VMEM((_D_CH, D), jnp.float32),
            pltpu.SemaphoreType.DMA,
            pltpu.SemaphoreType.DMA,
            pltpu.SemaphoreType.DMA,
        ],
    )
    return f(x2d, src_idx_g.reshape(NW, _D_NCH, _D_CH))


def _ffn_body(be_ref, bv_ref, xs_ref, w1_ref, w3_ref, w2_ref, wp_ref, *rest):
    ys_ref = rest[-1]
    i = pl.program_id(0)

    @pl.when(bv_ref[i] == 1)
    def _():
        xt = xs_ref[...]
        a = lax.dot_general(xt, w1_ref[0], (((1,), (1,)), ((), ())),
                            preferred_element_type=jnp.float32)
        g = lax.dot_general(xt, w3_ref[0], (((1,), (1,)), ((), ())),
                            preferred_element_type=jnp.float32)
        h = (a / (1.0 + jnp.exp(-a))) * g
        o = lax.dot_general(h, w2_ref[0], (((1,), (1,)), ((), ())),
                            preferred_element_type=jnp.float32)
        ys_ref[...] = o * wp_ref[0]


def _ffn_group(g, xs_g, wpad_g, be_g, bv_g, W1, W3, W2, ys_prev):
    in_specs = [
        pl.BlockSpec((BLOCK, D), lambda i, be, bv: (i, 0)),
        pl.BlockSpec((1, H, D), lambda i, be, bv: (be[i], 0, 0)),
        pl.BlockSpec((1, H, D), lambda i, be, bv: (be[i], 0, 0)),
        pl.BlockSpec((1, D, H), lambda i, be, bv: (be[i], 0, 0)),
        pl.BlockSpec((1, BLOCK, 1), lambda i, be, bv: (i, 0, 0)),
    ]
    args = [be_g, bv_g, xs_g, W1, W3, W2, wpad_g.reshape(NBG, BLOCK, 1)]
    aliases = {}
    if ys_prev is not None:
        in_specs.append(pl.BlockSpec(memory_space=pl.ANY))
        args.append(ys_prev)
        aliases = {7: 0}
    grid_spec = pltpu.PrefetchScalarGridSpec(
        num_scalar_prefetch=2,
        grid=(NBG,),
        in_specs=in_specs,
        out_specs=pl.BlockSpec((BLOCK, D),
                               lambda i, be, bv, g=g: (i + g * NBG, 0)),
    )
    return pl.pallas_call(
        _ffn_body,
        grid_spec=grid_spec,
        out_shape=jax.ShapeDtypeStruct((P_MAX, D), jnp.float32),
        input_output_aliases=aliases,
        compiler_params=pltpu.CompilerParams(
            dimension_semantics=("arbitrary",),
            vmem_limit_bytes=128 * 1024 * 1024,
        ),
    )(*args)


_C_CH = 32
_C_PER_W = T // NW
_C_NCH = _C_PER_W // _C_CH


def _combine_body(ys_hbm, q1_hbm, q2_hbm, out_hbm, q1_v, q2_v, bufa, bufb, sem):
    wid = lax.axis_index("s") * NC + lax.axis_index("c")
    pltpu.sync_copy(q1_hbm.at[wid], q1_v)
    pltpu.sync_copy(q2_hbm.at[wid], q2_v)
    base = wid * _C_PER_W
    for c in range(_C_NCH):
        pltpu.async_copy(ys_hbm.at[q1_v.at[c]], bufa, sem).wait()
        pltpu.async_copy(ys_hbm.at[q2_v.at[c]], bufb, sem).wait()

        def row(r, carry):
            def vec(v, carry2):
                sl = pl.ds(v * 16, 16)
                bufa[r, sl] = bufa[r, sl] + bufb[r, sl]
                return carry2
            return lax.fori_loop(0, D // 16, vec, carry)

        lax.fori_loop(0, _C_CH, row, 0)
        pltpu.sync_copy(bufa, out_hbm.at[pl.ds(base + c * _C_CH, _C_CH)])


def _combine(ys, q1, q2):
    mesh = plsc.VectorSubcoreMesh(core_axis_name="c", subcore_axis_name="s")
    f = pl.kernel(
        _combine_body,
        out_type=jax.ShapeDtypeStruct((T, D), jnp.float32),
        mesh=mesh,
        scratch_types=[
            pltpu.VMEM((_C_NCH, _C_CH), jnp.int32),
            pltpu.VMEM((_C_NCH, _C_CH), jnp.int32),
            pltpu.VMEM((_C_CH, D), jnp.float32),
            pltpu.VMEM((_C_CH, D), jnp.float32),
            pltpu.SemaphoreType.DMA,
        ],
    )
    return f(ys, q1.reshape(NW, _C_NCH, _C_CH), q2.reshape(NW, _C_NCH, _C_CH))


def kernel(x, Wg, W1, W3, W2):
    B, S, _ = x.shape
    x2d = x.reshape(T, D)
    ei, ws, hist3 = _router(x2d, Wg)
    src_idx, wpad, q1, q2, be, bvalid = _bookkeep(ei, ws, hist3)
    xs_groups = [_dispatch(x2d, src_idx[g * PG:(g + 1) * PG]) for g in range(G)]
    ys = None
    for g in range(G):
        ys = _ffn_group(
            g, xs_groups[g], wpad[g * PG:(g + 1) * PG],
            be[g * NBG:(g + 1) * NBG], bvalid[g * NBG:(g + 1) * NBG],
            W1, W3, W2, ys)
    out = _combine(ys, q1, q2)
    return out.reshape(B, S, D)


# speedup vs baseline: 1.4836x; 1.4654x over previous
import functools

import jax
import jax.numpy as jnp
from jax import lax
from jax.experimental import pallas as pl
from jax.experimental.pallas import tpu as pltpu
from jax.experimental.pallas import tpu_sc as plsc

D = 1024
H = 2048
E = 64
T = 4096

BLOCK = 128
NB = 2 * T // BLOCK + E
P_MAX = NB * BLOCK

NC, NS = 2, 16
NW = NC * NS

RT = 256


NT = T // RT


def _router_body(x_ref, wg_ref, ei_ref, ws_ref, hist_ref):
    s = lax.dot_general(x_ref[...], wg_ref[...], (((1,), (1,)), ((), ())),
                        preferred_element_type=jnp.float32)
    ecol = lax.broadcasted_iota(jnp.int32, (RT, E), 1)
    m1 = jnp.max(s, axis=1, keepdims=True)
    i1 = jnp.min(jnp.where(s == m1, ecol, E), axis=1, keepdims=True)
    s2 = jnp.where(ecol == i1, -1e30, s)
    m2 = jnp.max(s2, axis=1, keepdims=True)
    i2 = jnp.min(jnp.where(s2 == m2, ecol, E), axis=1, keepdims=True)
    w1 = 1.0 / (1.0 + jnp.exp(m2 - m1))
    ei_ref[...] = jnp.concatenate([i1, i2], axis=1)
    ws_ref[...] = jnp.concatenate([w1, 1.0 - w1], axis=1)
    cnt = jnp.sum(((ecol == i1) | (ecol == i2)).astype(jnp.float32), axis=0,
                  keepdims=True)
    hist_ref[...] = jnp.concatenate(
        [cnt, jnp.zeros((7, E), jnp.float32)], axis=0)[None]


def _router(x2d, Wg):
    return pl.pallas_call(
        _router_body,
        grid=(NT,),
        in_specs=[
            pl.BlockSpec((RT, D), lambda i: (i, 0)),
            pl.BlockSpec((E, D), lambda i: (0, 0)),
        ],
        out_specs=[
            pl.BlockSpec((RT, 2), lambda i: (i, 0)),
            pl.BlockSpec((RT, 2), lambda i: (i, 0)),
            pl.BlockSpec((1, 8, E), lambda i: (i, 0, 0)),
        ],
        out_shape=[
            jax.ShapeDtypeStruct((T, 2), jnp.int32),
            jax.ShapeDtypeStruct((T, 2), jnp.float32),
            jax.ShapeDtypeStruct((NT, 8, E), jnp.float32),
        ],
    )(x2d, Wg)


def _rank_body(ei_ref, cb_ref, qp_ref):
    e0 = ei_ref[:, 0:1]
    e1 = ei_ref[:, 1:2]
    ecol = lax.broadcasted_iota(jnp.int32, (RT, E), 1)
    oh0 = (ecol == e0).astype(jnp.float32)
    oh1 = (ecol == e1).astype(jnp.float32)
    rowcnt = oh0 + oh1
    ri = lax.broadcasted_iota(jnp.int32, (RT, RT), 0)
    ci = lax.broadcasted_iota(jnp.int32, (RT, RT), 1)
    ltri = (ci < ri).astype(jnp.float32)
    prior = lax.dot_general(ltri, rowcnt, (((1,), (0,)), ((), ())),
                            preferred_element_type=jnp.float32)
    base = cb_ref[0, 0:1, :] + prior
    q0 = jnp.sum(base * oh0, axis=1, keepdims=True)
    q1 = jnp.sum(base * oh1, axis=1, keepdims=True) + (e0 == e1)
    qp_ref[...] = jnp.concatenate([q0, q1], axis=1).astype(jnp.int32)


def _rank(ei, cbase):
    return pl.pallas_call(
        _rank_body,
        grid=(NT,),
        in_specs=[
            pl.BlockSpec((RT, 2), lambda i: (i, 0)),
            pl.BlockSpec((1, 8, E), lambda i: (i, 0, 0)),
        ],
        out_specs=pl.BlockSpec((RT, 2), lambda i: (i, 0)),
        out_shape=jax.ShapeDtypeStruct((T, 2), jnp.int32),
    )(ei, cbase)


def _bookkeep(ei, ws, hist3):
    i32 = jnp.int32
    hist = hist3[:, 0, :]
    counts = jnp.sum(hist, axis=0).astype(i32)
    tilebase = jnp.cumsum(hist, axis=0) - hist
    nblk = (counts + BLOCK - 1) // BLOCK
    pstart = (jnp.cumsum(nblk) - nblk) * BLOCK
    cbase = pstart.astype(jnp.float32)[None, :] + tilebase
    cbase3 = jnp.broadcast_to(cbase[:, None, :], (NT, 8, E))
    qp = _rank(ei, cbase3)
    qf = qp.reshape(-1)
    wpad = jnp.zeros((P_MAX,), jnp.float32).at[qf].set(ws.reshape(-1))
    bids = jnp.arange(NB, dtype=i32)
    be = jnp.clip(
        jnp.searchsorted(pstart, bids * BLOCK, side="right").astype(i32) - 1,
        0, E - 1)
    bvalid = (bids * BLOCK < jnp.sum(nblk) * BLOCK).astype(i32)
    return wpad, qp[:, 0], qp[:, 1], be, bvalid


G = 1
PG = P_MAX // G
NBG = NB // G

_D_CH = 32
_D_PER_W = T // NW
_D_NCH = _D_PER_W // _D_CH


def _dispatch_body(x_hbm, q1_hbm, q2_hbm, xs_hbm, q1_v, q2_v, b0, b1, s0, s1):
    wid = lax.axis_index("s") * NC + lax.axis_index("c")
    pltpu.sync_copy(q1_hbm.at[wid], q1_v)
    pltpu.sync_copy(q2_hbm.at[wid], q2_v)
    base = wid * _D_PER_W
    bufs, sems, qvs = (b0, b1), (s0, s1), (q1_v, q2_v)

    def wait_scatters(c, b):
        for qv in qvs:
            pltpu.make_async_copy(bufs[b], xs_hbm.at[qv.at[c]], sems[b]).wait()

    for c in range(_D_NCH):
        b = c % 2
        if c >= 2:
            wait_scatters(c - 2, b)
        pltpu.sync_copy(x_hbm.at[pl.ds(base + c * _D_CH, _D_CH)], bufs[b])
        for qv in qvs:
            pltpu.async_copy(bufs[b], xs_hbm.at[qv.at[c]], sems[b])
    for c in range(max(_D_NCH - 2, 0), _D_NCH):
        wait_scatters(c, c % 2)


def _dispatch(x2d, q1, q2):
    mesh = plsc.VectorSubcoreMesh(core_axis_name="c", subcore_axis_name="s")
    f = pl.kernel(
        _dispatch_body,
        out_type=jax.ShapeDtypeStruct((P_MAX, D), jnp.float32),
        mesh=mesh,
        scratch_types=[
            pltpu.VMEM((_D_NCH, _D_CH), jnp.int32),
            pltpu.VMEM((_D_NCH, _D_CH), jnp.int32),
            pltpu.VMEM((_D_CH, D), jnp.float32),
            pltpu.VMEM((_D_CH, D), jnp.float32),
            pltpu.SemaphoreType.DMA,
            pltpu.SemaphoreType.DMA,
        ],
    )
    return f(x2d, q1.reshape(NW, _D_NCH, _D_CH), q2.reshape(NW, _D_NCH, _D_CH))


def _ffn_body(be_ref, bv_ref, xs_ref, w1_ref, w3_ref, w2_ref, wp_ref, *rest):
    ys_ref = rest[-1]
    i = pl.program_id(0)

    @pl.when(bv_ref[i] == 1)
    def _():
        xt = xs_ref[...]
        a = lax.dot_general(xt, w1_ref[0], (((1,), (1,)), ((), ())),
                            preferred_element_type=jnp.float32)
        g = lax.dot_general(xt, w3_ref[0], (((1,), (1,)), ((), ())),
                            preferred_element_type=jnp.float32)
        h = (a / (1.0 + jnp.exp(-a))) * g
        o = lax.dot_general(h, w2_ref[0], (((1,), (1,)), ((), ())),
                            preferred_element_type=jnp.float32)
        ys_ref[...] = o * wp_ref[0]


def _ffn_group(g, xs_g, wpad_g, be_g, bv_g, W1, W3, W2, ys_prev):
    in_specs = [
        pl.BlockSpec((BLOCK, D), lambda i, be, bv: (i, 0)),
        pl.BlockSpec((1, H, D), lambda i, be, bv: (be[i], 0, 0)),
        pl.BlockSpec((1, H, D), lambda i, be, bv: (be[i], 0, 0)),
        pl.BlockSpec((1, D, H), lambda i, be, bv: (be[i], 0, 0)),
        pl.BlockSpec((1, BLOCK, 1), lambda i, be, bv: (i, 0, 0)),
    ]
    args = [be_g, bv_g, xs_g, W1, W3, W2, wpad_g.reshape(NBG, BLOCK, 1)]
    aliases = {}
    if ys_prev is not None:
        in_specs.append(pl.BlockSpec(memory_space=pl.ANY))
        args.append(ys_prev)
        aliases = {7: 0}
    grid_spec = pltpu.PrefetchScalarGridSpec(
        num_scalar_prefetch=2,
        grid=(NBG,),
        in_specs=in_specs,
        out_specs=pl.BlockSpec((BLOCK, D),
                               lambda i, be, bv, g=g: (i + g * NBG, 0)),
    )
    return pl.pallas_call(
        _ffn_body,
        grid_spec=grid_spec,
        out_shape=jax.ShapeDtypeStruct((P_MAX, D), jnp.float32),
        input_output_aliases=aliases,
        compiler_params=pltpu.CompilerParams(
            dimension_semantics=("arbitrary",),
            vmem_limit_bytes=128 * 1024 * 1024,
        ),
    )(*args)


_C_CH = 32
_C_PER_W = T // NW
_C_NCH = _C_PER_W // _C_CH


def _combine_body(ys_hbm, q1_hbm, q2_hbm, out_hbm, q1_v, q2_v, bufa, bufb, sem):
    wid = lax.axis_index("s") * NC + lax.axis_index("c")
    pltpu.sync_copy(q1_hbm.at[wid], q1_v)
    pltpu.sync_copy(q2_hbm.at[wid], q2_v)
    base = wid * _C_PER_W
    for c in range(_C_NCH):
        pltpu.async_copy(ys_hbm.at[q1_v.at[c]], bufa, sem).wait()
        pltpu.async_copy(ys_hbm.at[q2_v.at[c]], bufb, sem).wait()

        def row(r, carry):
            def vec(v, carry2):
                sl = pl.ds(v * 16, 16)
                bufa[r, sl] = bufa[r, sl] + bufb[r, sl]
                return carry2
            return lax.fori_loop(0, D // 16, vec, carry)

        lax.fori_loop(0, _C_CH, row, 0)
        pltpu.sync_copy(bufa, out_hbm.at[pl.ds(base + c * _C_CH, _C_CH)])


def _combine(ys, q1, q2):
    mesh = plsc.VectorSubcoreMesh(core_axis_name="c", subcore_axis_name="s")
    f = pl.kernel(
        _combine_body,
        out_type=jax.ShapeDtypeStruct((T, D), jnp.float32),
        mesh=mesh,
        scratch_types=[
            pltpu.VMEM((_C_NCH, _C_CH), jnp.int32),
            pltpu.VMEM((_C_NCH, _C_CH), jnp.int32),
            pltpu.VMEM((_C_CH, D), jnp.float32),
            pltpu.VMEM((_C_CH, D), jnp.float32),
            pltpu.SemaphoreType.DMA,
        ],
    )
    return f(ys, q1.reshape(NW, _C_NCH, _C_CH), q2.reshape(NW, _C_NCH, _C_CH))


def kernel(x, Wg, W1, W3, W2):
    B, S, _ = x.shape
    x2d = x.reshape(T, D)
    ei, ws, hist3 = _router(x2d, Wg)
    wpad, q1, q2, be, bvalid = _bookkeep(ei, ws, hist3)
    xs = _dispatch(x2d, q1, q2)
    ys = _ffn_group(0, xs, wpad, be, bvalid, W1, W3, W2, None)
    out = _combine(ys, q1, q2)
    return out.reshape(B, S, D)


# speedup vs baseline: 1.5065x; 1.0155x over previous
import functools

import jax
import jax.numpy as jnp
from jax import lax
from jax.experimental import pallas as pl
from jax.experimental.pallas import tpu as pltpu
from jax.experimental.pallas import tpu_sc as plsc

D = 1024
H = 2048
E = 64
T = 4096

BLOCK = 128
NB = 2 * T // BLOCK + E
P_MAX = NB * BLOCK

NC, NS = 2, 16
NW = NC * NS

RT = 256


NT = T // RT


def _router_body(x_ref, wg_ref, ei_ref, ws_ref, hist_ref):
    s = lax.dot_general(x_ref[...], wg_ref[...], (((1,), (1,)), ((), ())),
                        preferred_element_type=jnp.float32)
    ecol = lax.broadcasted_iota(jnp.int32, (RT, E), 1)
    m1 = jnp.max(s, axis=1, keepdims=True)
    i1 = jnp.min(jnp.where(s == m1, ecol, E), axis=1, keepdims=True)
    s2 = jnp.where(ecol == i1, -1e30, s)
    m2 = jnp.max(s2, axis=1, keepdims=True)
    i2 = jnp.min(jnp.where(s2 == m2, ecol, E), axis=1, keepdims=True)
    w1 = 1.0 / (1.0 + jnp.exp(m2 - m1))
    ei_ref[...] = jnp.concatenate([i1, i2], axis=1)
    ws_ref[...] = jnp.concatenate([w1, 1.0 - w1], axis=1)
    cnt = jnp.sum(((ecol == i1) | (ecol == i2)).astype(jnp.float32), axis=0,
                  keepdims=True)
    hist_ref[...] = jnp.concatenate(
        [cnt, jnp.zeros((7, E), jnp.float32)], axis=0)[None]


def _router(x2d, Wg):
    return pl.pallas_call(
        _router_body,
        grid=(NT,),
        in_specs=[
            pl.BlockSpec((RT, D), lambda i: (i, 0)),
            pl.BlockSpec((E, D), lambda i: (0, 0)),
        ],
        out_specs=[
            pl.BlockSpec((RT, 2), lambda i: (i, 0)),
            pl.BlockSpec((RT, 2), lambda i: (i, 0)),
            pl.BlockSpec((1, 8, E), lambda i: (i, 0, 0)),
        ],
        out_shape=[
            jax.ShapeDtypeStruct((T, 2), jnp.int32),
            jax.ShapeDtypeStruct((T, 2), jnp.float32),
            jax.ShapeDtypeStruct((NT, 8, E), jnp.float32),
        ],
    )(x2d, Wg)


def _rank_body(ei_ref, cb_ref, qp_ref):
    e0 = ei_ref[:, 0:1]
    e1 = ei_ref[:, 1:2]
    ecol = lax.broadcasted_iota(jnp.int32, (RT, E), 1)
    oh0 = (ecol == e0).astype(jnp.float32)
    oh1 = (ecol == e1).astype(jnp.float32)
    rowcnt = oh0 + oh1
    ri = lax.broadcasted_iota(jnp.int32, (RT, RT), 0)
    ci = lax.broadcasted_iota(jnp.int32, (RT, RT), 1)
    ltri = (ci < ri).astype(jnp.float32)
    prior = lax.dot_general(ltri, rowcnt, (((1,), (0,)), ((), ())),
                            preferred_element_type=jnp.float32)
    base = cb_ref[0, 0:1, :] + prior
    q0 = jnp.sum(base * oh0, axis=1, keepdims=True)
    q1 = jnp.sum(base * oh1, axis=1, keepdims=True) + (e0 == e1)
    qp_ref[...] = jnp.concatenate([q0, q1], axis=1).astype(jnp.int32)


def _rank(ei, cbase):
    return pl.pallas_call(
        _rank_body,
        grid=(NT,),
        in_specs=[
            pl.BlockSpec((RT, 2), lambda i: (i, 0)),
            pl.BlockSpec((1, 8, E), lambda i: (i, 0, 0)),
        ],
        out_specs=pl.BlockSpec((RT, 2), lambda i: (i, 0)),
        out_shape=jax.ShapeDtypeStruct((T, 2), jnp.int32),
    )(ei, cbase)


def _bookkeep(ei, ws, hist3):
    i32 = jnp.int32
    hist = hist3[:, 0, :]
    counts = jnp.sum(hist, axis=0).astype(i32)
    tilebase = jnp.cumsum(hist, axis=0) - hist
    nblk = (counts + BLOCK - 1) // BLOCK
    pstart = (jnp.cumsum(nblk) - nblk) * BLOCK
    cbase = pstart.astype(jnp.float32)[None, :] + tilebase
    cbase3 = jnp.broadcast_to(cbase[:, None, :], (NT, 8, E))
    qp = _rank(ei, cbase3)
    qf = qp.reshape(-1)
    wpad = jnp.zeros((P_MAX,), jnp.float32).at[qf].set(ws.reshape(-1))
    bids = jnp.arange(NB, dtype=i32)
    be = jnp.clip(
        jnp.searchsorted(pstart, bids * BLOCK, side="right").astype(i32) - 1,
        0, E - 1)
    bvalid = (bids * BLOCK < jnp.sum(nblk) * BLOCK).astype(i32)
    return wpad, qp[:, 0], qp[:, 1], be, bvalid


G = 1
PG = P_MAX // G
NBG = NB // G

_D_CH = 32
_D_PER_W = T // NW
_D_NCH = _D_PER_W // _D_CH


def _dispatch_body(x_hbm, q1_hbm, q2_hbm, xs_hbm, q1_v, q2_v, b0, b1, s0, s1):
    wid = lax.axis_index("s") * NC + lax.axis_index("c")
    pltpu.sync_copy(q1_hbm.at[wid], q1_v)
    pltpu.sync_copy(q2_hbm.at[wid], q2_v)
    base = wid * _D_PER_W
    bufs, sems, qvs = (b0, b1), (s0, s1), (q1_v, q2_v)

    def wait_scatters(c, b):
        for qv in qvs:
            pltpu.make_async_copy(bufs[b], xs_hbm.at[qv.at[c]], sems[b]).wait()

    for c in range(_D_NCH):
        b = c % 2
        if c >= 2:
            wait_scatters(c - 2, b)
        pltpu.sync_copy(x_hbm.at[pl.ds(base + c * _D_CH, _D_CH)], bufs[b])
        for qv in qvs:
            pltpu.async_copy(bufs[b], xs_hbm.at[qv.at[c]], sems[b])
    for c in range(max(_D_NCH - 2, 0), _D_NCH):
        wait_scatters(c, c % 2)


def _dispatch(x2d, q1, q2):
    mesh = plsc.VectorSubcoreMesh(core_axis_name="c", subcore_axis_name="s")
    f = pl.kernel(
        _dispatch_body,
        out_type=jax.ShapeDtypeStruct((P_MAX, D), jnp.float32),
        mesh=mesh,
        scratch_types=[
            pltpu.VMEM((_D_NCH, _D_CH), jnp.int32),
            pltpu.VMEM((_D_NCH, _D_CH), jnp.int32),
            pltpu.VMEM((_D_CH, D), jnp.float32),
            pltpu.VMEM((_D_CH, D), jnp.float32),
            pltpu.SemaphoreType.DMA,
            pltpu.SemaphoreType.DMA,
        ],
    )
    return f(x2d, q1.reshape(NW, _D_NCH, _D_CH), q2.reshape(NW, _D_NCH, _D_CH))


def _ffn_body(be_ref, bv_ref, xs_ref, w1_ref, w3_ref, w2_ref, wp_ref, *rest):
    ys_ref = rest[-1]
    i = pl.program_id(0)

    @pl.when(bv_ref[i] == 1)
    def _():
        xt = xs_ref[...]
        a = lax.dot_general(xt, w1_ref[0], (((1,), (1,)), ((), ())),
                            preferred_element_type=jnp.float32)
        g = lax.dot_general(xt, w3_ref[0], (((1,), (1,)), ((), ())),
                            preferred_element_type=jnp.float32)
        h = (a / (1.0 + jnp.exp(-a))) * g
        o = lax.dot_general(h, w2_ref[0], (((1,), (1,)), ((), ())),
                            preferred_element_type=jnp.float32)
        ys_ref[...] = o * wp_ref[0]


def _ffn_group(g, xs_g, wpad_g, be_g, bv_g, W1, W3, W2, ys_prev):
    in_specs = [
        pl.BlockSpec((BLOCK, D), lambda i, be, bv: (i, 0)),
        pl.BlockSpec((1, H, D), lambda i, be, bv: (be[i], 0, 0)),
        pl.BlockSpec((1, H, D), lambda i, be, bv: (be[i], 0, 0)),
        pl.BlockSpec((1, D, H), lambda i, be, bv: (be[i], 0, 0)),
        pl.BlockSpec((1, BLOCK, 1), lambda i, be, bv: (i, 0, 0)),
    ]
    args = [be_g, bv_g, xs_g, W1, W3, W2, wpad_g.reshape(NBG, BLOCK, 1)]
    aliases = {}
    if ys_prev is not None:
        in_specs.append(pl.BlockSpec(memory_space=pl.ANY))
        args.append(ys_prev)
        aliases = {7: 0}
    grid_spec = pltpu.PrefetchScalarGridSpec(
        num_scalar_prefetch=2,
        grid=(NBG,),
        in_specs=in_specs,
        out_specs=pl.BlockSpec((BLOCK, D),
                               lambda i, be, bv, g=g: (i + g * NBG, 0)),
    )
    return pl.pallas_call(
        _ffn_body,
        grid_spec=grid_spec,
        out_shape=jax.ShapeDtypeStruct((P_MAX, D), jnp.float32),
        input_output_aliases=aliases,
        compiler_params=pltpu.CompilerParams(
            dimension_semantics=("arbitrary",),
            vmem_limit_bytes=128 * 1024 * 1024,
        ),
    )(*args)


_C_CH = 16
_C_PER_W = T // NW
_C_NCH = _C_PER_W // _C_CH


def _combine_body(ys_hbm, q1_hbm, q2_hbm, out_hbm, q1_v, q2_v,
                  a0, b0, a1, b1, s0, s1):
    wid = lax.axis_index("s") * NC + lax.axis_index("c")
    pltpu.sync_copy(q1_hbm.at[wid], q1_v)
    pltpu.sync_copy(q2_hbm.at[wid], q2_v)
    base = wid * _C_PER_W
    abufs, bbufs, sems = (a0, a1), (b0, b1), (s0, s1)

    def issue(c, k):
        pltpu.async_copy(ys_hbm.at[q1_v.at[c]], abufs[k], sems[k])
        pltpu.async_copy(ys_hbm.at[q2_v.at[c]], bbufs[k], sems[k])

    def drain(c, k):
        pltpu.make_async_copy(ys_hbm.at[q1_v.at[c]], abufs[k], sems[k]).wait()
        pltpu.make_async_copy(ys_hbm.at[q2_v.at[c]], bbufs[k], sems[k]).wait()

    issue(0, 0)
    for c in range(_C_NCH):
        k = c % 2
        drain(c, k)
        if c + 1 < _C_NCH:
            issue(c + 1, 1 - k)
        bufa, bufb = abufs[k], bbufs[k]

        def row(r, carry):
            def vec(v, carry2):
                sl = pl.ds(v * 16, 16)
                bufa[r, sl] = bufa[r, sl] + bufb[r, sl]
                return carry2
            return lax.fori_loop(0, D // 16, vec, carry)

        lax.fori_loop(0, _C_CH, row, 0)
        pltpu.sync_copy(bufa, out_hbm.at[pl.ds(base + c * _C_CH, _C_CH)])


def _combine(ys, q1, q2):
    mesh = plsc.VectorSubcoreMesh(core_axis_name="c", subcore_axis_name="s")
    f = pl.kernel(
        _combine_body,
        out_type=jax.ShapeDtypeStruct((T, D), jnp.float32),
        mesh=mesh,
        scratch_types=[
            pltpu.VMEM((_C_NCH, _C_CH), jnp.int32),
            pltpu.VMEM((_C_NCH, _C_CH), jnp.int32),
            pltpu.VMEM((_C_CH, D), jnp.float32),
            pltpu.VMEM((_C_CH, D), jnp.float32),
            pltpu.VMEM((_C_CH, D), jnp.float32),
            pltpu.VMEM((_C_CH, D), jnp.float32),
            pltpu.SemaphoreType.DMA,
            pltpu.SemaphoreType.DMA,
        ],
    )
    return f(ys, q1.reshape(NW, _C_NCH, _C_CH), q2.reshape(NW, _C_NCH, _C_CH))


def kernel(x, Wg, W1, W3, W2):
    B, S, _ = x.shape
    x2d = x.reshape(T, D)
    ei, ws, hist3 = _router(x2d, Wg)
    wpad, q1, q2, be, bvalid = _bookkeep(ei, ws, hist3)
    xs = _dispatch(x2d, q1, q2)
    ys = _ffn_group(0, xs, wpad, be, bvalid, W1, W3, W2, None)
    out = _combine(ys, q1, q2)
    return out.reshape(B, S, D)


# speedup vs baseline: 1.5076x; 1.0007x over previous
import jax
import jax.numpy as jnp
from jax import lax
from jax.experimental import pallas as pl
from jax.experimental.pallas import tpu as pltpu
from jax.experimental.pallas import tpu_sc as plsc

D = 1024
H = 2048
E = 64
T = 4096

BLOCK = 128
NB = 2 * T // BLOCK + E
P_MAX = NB * BLOCK

NC, NS = 2, 16
NW = NC * NS

RT = 256


NT = T // RT


def _router_body(x_ref, wg_ref, ei_ref, ws_ref, hist_ref):
    s = lax.dot_general(x_ref[...], wg_ref[...], (((1,), (1,)), ((), ())),
                        preferred_element_type=jnp.float32)
    ecol = lax.broadcasted_iota(jnp.int32, (RT, E), 1)
    m1 = jnp.max(s, axis=1, keepdims=True)
    i1 = jnp.min(jnp.where(s == m1, ecol, E), axis=1, keepdims=True)
    s2 = jnp.where(ecol == i1, -1e30, s)
    m2 = jnp.max(s2, axis=1, keepdims=True)
    i2 = jnp.min(jnp.where(s2 == m2, ecol, E), axis=1, keepdims=True)
    w1 = 1.0 / (1.0 + jnp.exp(m2 - m1))
    ei_ref[...] = jnp.concatenate([i1, i2], axis=1)
    ws_ref[...] = jnp.concatenate([w1, 1.0 - w1], axis=1)
    cnt = jnp.sum(((ecol == i1) | (ecol == i2)).astype(jnp.float32), axis=0,
                  keepdims=True)
    hist_ref[...] = jnp.concatenate(
        [cnt, jnp.zeros((7, E), jnp.float32)], axis=0)[None]


def _router(x2d, Wg):
    return pl.pallas_call(
        _router_body,
        grid=(NT,),
        in_specs=[
            pl.BlockSpec((RT, D), lambda i: (i, 0)),
            pl.BlockSpec((E, D), lambda i: (0, 0)),
        ],
        out_specs=[
            pl.BlockSpec((RT, 2), lambda i: (i, 0)),
            pl.BlockSpec((RT, 2), lambda i: (i, 0)),
            pl.BlockSpec((1, 8, E), lambda i: (i, 0, 0)),
        ],
        out_shape=[
            jax.ShapeDtypeStruct((T, 2), jnp.int32),
            jax.ShapeDtypeStruct((T, 2), jnp.float32),
            jax.ShapeDtypeStruct((NT, 8, E), jnp.float32),
        ],
    )(x2d, Wg)


def _rank_body(ei_ref, cb_ref, qp_ref):
    e0 = ei_ref[:, 0:1]
    e1 = ei_ref[:, 1:2]
    ecol = lax.broadcasted_iota(jnp.int32, (RT, E), 1)
    oh0 = (ecol == e0).astype(jnp.float32)
    oh1 = (ecol == e1).astype(jnp.float32)
    rowcnt = oh0 + oh1
    ri = lax.broadcasted_iota(jnp.int32, (RT, RT), 0)
    ci = lax.broadcasted_iota(jnp.int32, (RT, RT), 1)
    ltri = (ci < ri).astype(jnp.float32)
    prior = lax.dot_general(ltri, rowcnt, (((1,), (0,)), ((), ())),
                            preferred_element_type=jnp.float32)
    base = cb_ref[0, 0:1, :] + prior
    q0 = jnp.sum(base * oh0, axis=1, keepdims=True)
    q1 = jnp.sum(base * oh1, axis=1, keepdims=True) + (e0 == e1)
    qp_ref[...] = jnp.concatenate([q0, q1], axis=1).astype(jnp.int32)


def _rank(ei, cbase):
    return pl.pallas_call(
        _rank_body,
        grid=(NT,),
        in_specs=[
            pl.BlockSpec((RT, 2), lambda i: (i, 0)),
            pl.BlockSpec((1, 8, E), lambda i: (i, 0, 0)),
        ],
        out_specs=pl.BlockSpec((RT, 2), lambda i: (i, 0)),
        out_shape=jax.ShapeDtypeStruct((T, 2), jnp.int32),
    )(ei, cbase)


def _bookkeep(ei, ws, hist3):
    i32 = jnp.int32
    hist = hist3[:, 0, :]
    counts = jnp.sum(hist, axis=0).astype(i32)
    tilebase = jnp.cumsum(hist, axis=0) - hist
    nblk = (counts + BLOCK - 1) // BLOCK
    pstart = (jnp.cumsum(nblk) - nblk) * BLOCK
    cbase = pstart.astype(jnp.float32)[None, :] + tilebase
    cbase3 = jnp.broadcast_to(cbase[:, None, :], (NT, 8, E))
    qp = _rank(ei, cbase3)
    qf = qp.reshape(-1)
    wpad = jnp.zeros((P_MAX,), jnp.float32).at[qf].set(ws.reshape(-1))
    bids = jnp.arange(NB, dtype=i32)
    be = jnp.clip(
        jnp.searchsorted(pstart, bids * BLOCK, side="right").astype(i32) - 1,
        0, E - 1)
    bvalid = (bids * BLOCK < jnp.sum(nblk) * BLOCK).astype(i32)
    return wpad, qp[:, 0], qp[:, 1], be, bvalid


NBG = NB

_D_CH = 32
_D_PER_W = T // NW
_D_NCH = _D_PER_W // _D_CH


def _dispatch_body(x_hbm, q1_hbm, q2_hbm, xs_hbm, q1_v, q2_v, b0, b1, s0, s1):
    wid = lax.axis_index("s") * NC + lax.axis_index("c")
    pltpu.sync_copy(q1_hbm.at[wid], q1_v)
    pltpu.sync_copy(q2_hbm.at[wid], q2_v)
    base = wid * _D_PER_W
    bufs, sems, qvs = (b0, b1), (s0, s1), (q1_v, q2_v)

    def wait_scatters(c, b):
        for qv in qvs:
            pltpu.make_async_copy(bufs[b], xs_hbm.at[qv.at[c]], sems[b]).wait()

    for c in range(_D_NCH):
        b = c % 2
        if c >= 2:
            wait_scatters(c - 2, b)
        pltpu.sync_copy(x_hbm.at[pl.ds(base + c * _D_CH, _D_CH)], bufs[b])
        for qv in qvs:
            pltpu.async_copy(bufs[b], xs_hbm.at[qv.at[c]], sems[b])
    for c in range(max(_D_NCH - 2, 0), _D_NCH):
        wait_scatters(c, c % 2)


def _dispatch(x2d, q1, q2):
    mesh = plsc.VectorSubcoreMesh(core_axis_name="c", subcore_axis_name="s")
    f = pl.kernel(
        _dispatch_body,
        out_type=jax.ShapeDtypeStruct((P_MAX, D), jnp.float32),
        mesh=mesh,
        scratch_types=[
            pltpu.VMEM((_D_NCH, _D_CH), jnp.int32),
            pltpu.VMEM((_D_NCH, _D_CH), jnp.int32),
            pltpu.VMEM((_D_CH, D), jnp.float32),
            pltpu.VMEM((_D_CH, D), jnp.float32),
            pltpu.SemaphoreType.DMA,
            pltpu.SemaphoreType.DMA,
        ],
    )
    return f(x2d, q1.reshape(NW, _D_NCH, _D_CH), q2.reshape(NW, _D_NCH, _D_CH))


def _ffn_body(be_ref, bv_ref, xs_ref, w1_ref, w3_ref, w2_ref, wp_ref, *rest):
    ys_ref = rest[-1]
    i = pl.program_id(0)

    @pl.when(bv_ref[i] == 1)
    def _():
        xt = xs_ref[...]
        a = lax.dot_general(xt, w1_ref[0], (((1,), (1,)), ((), ())),
                            preferred_element_type=jnp.float32)
        g = lax.dot_general(xt, w3_ref[0], (((1,), (1,)), ((), ())),
                            preferred_element_type=jnp.float32)
        h = (a / (1.0 + jnp.exp(-a))) * g
        o = lax.dot_general(h, w2_ref[0], (((1,), (1,)), ((), ())),
                            preferred_element_type=jnp.float32)
        ys_ref[...] = o * wp_ref[0]


def _ffn_group(g, xs_g, wpad_g, be_g, bv_g, W1, W3, W2, ys_prev):
    in_specs = [
        pl.BlockSpec((BLOCK, D), lambda i, be, bv: (i, 0)),
        pl.BlockSpec((1, H, D), lambda i, be, bv: (be[i], 0, 0)),
        pl.BlockSpec((1, H, D), lambda i, be, bv: (be[i], 0, 0)),
        pl.BlockSpec((1, D, H), lambda i, be, bv: (be[i], 0, 0)),
        pl.BlockSpec((1, BLOCK, 1), lambda i, be, bv: (i, 0, 0)),
    ]
    args = [be_g, bv_g, xs_g, W1, W3, W2, wpad_g.reshape(NBG, BLOCK, 1)]
    aliases = {}
    if ys_prev is not None:
        in_specs.append(pl.BlockSpec(memory_space=pl.ANY))
        args.append(ys_prev)
        aliases = {7: 0}
    grid_spec = pltpu.PrefetchScalarGridSpec(
        num_scalar_prefetch=2,
        grid=(NBG,),
        in_specs=in_specs,
        out_specs=pl.BlockSpec((BLOCK, D),
                               lambda i, be, bv, g=g: (i + g * NBG, 0)),
    )
    return pl.pallas_call(
        _ffn_body,
        grid_spec=grid_spec,
        out_shape=jax.ShapeDtypeStruct((P_MAX, D), jnp.float32),
        input_output_aliases=aliases,
        compiler_params=pltpu.CompilerParams(
            dimension_semantics=("arbitrary",),
            vmem_limit_bytes=128 * 1024 * 1024,
        ),
    )(*args)


_C_CH = 16
_C_PER_W = T // NW
_C_NCH = _C_PER_W // _C_CH


def _combine_body(ys_hbm, q1_hbm, q2_hbm, out_hbm, q1_v, q2_v,
                  a0, b0, a1, b1, s0, s1):
    wid = lax.axis_index("s") * NC + lax.axis_index("c")
    pltpu.sync_copy(q1_hbm.at[wid], q1_v)
    pltpu.sync_copy(q2_hbm.at[wid], q2_v)
    base = wid * _C_PER_W
    abufs, bbufs, sems = (a0, a1), (b0, b1), (s0, s1)

    def issue(c, k):
        pltpu.async_copy(ys_hbm.at[q1_v.at[c]], abufs[k], sems[k])
        pltpu.async_copy(ys_hbm.at[q2_v.at[c]], bbufs[k], sems[k])

    def drain(c, k):
        pltpu.make_async_copy(ys_hbm.at[q1_v.at[c]], abufs[k], sems[k]).wait()
        pltpu.make_async_copy(ys_hbm.at[q2_v.at[c]], bbufs[k], sems[k]).wait()

    issue(0, 0)
    for c in range(_C_NCH):
        k = c % 2
        drain(c, k)
        if c + 1 < _C_NCH:
            issue(c + 1, 1 - k)
        bufa, bufb = abufs[k], bbufs[k]

        def row(r, carry):
            def vec(v, carry2):
                sl = pl.ds(v * 16, 16)
                bufa[r, sl] = bufa[r, sl] + bufb[r, sl]
                return carry2
            return lax.fori_loop(0, D // 16, vec, carry)

        lax.fori_loop(0, _C_CH, row, 0)
        pltpu.sync_copy(bufa, out_hbm.at[pl.ds(base + c * _C_CH, _C_CH)])


def _combine(ys, q1, q2):
    mesh = plsc.VectorSubcoreMesh(core_axis_name="c", subcore_axis_name="s")
    f = pl.kernel(
        _combine_body,
        out_type=jax.ShapeDtypeStruct((T, D), jnp.float32),
        mesh=mesh,
        scratch_types=[
            pltpu.VMEM((_C_NCH, _C_CH), jnp.int32),
            pltpu.VMEM((_C_NCH, _C_CH), jnp.int32),
            pltpu.VMEM((_C_CH, D), jnp.float32),
            pltpu.VMEM((_C_CH, D), jnp.float32),
            pltpu.VMEM((_C_CH, D), jnp.float32),
            pltpu.VMEM((_C_CH, D), jnp.float32),
            pltpu.SemaphoreType.DMA,
            pltpu.SemaphoreType.DMA,
        ],
    )
    return f(ys, q1.reshape(NW, _C_NCH, _C_CH), q2.reshape(NW, _C_NCH, _C_CH))


def kernel(x, Wg, W1, W3, W2):
    B, S, _ = x.shape
    x2d = x.reshape(T, D)
    ei, ws, hist3 = _router(x2d, Wg)
    wpad, q1, q2, be, bvalid = _bookkeep(ei, ws, hist3)
    xs = _dispatch(x2d, q1, q2)
    ys = _ffn_group(0, xs, wpad, be, bvalid, W1, W3, W2, None)
    out = _combine(ys, q1, q2)
    return out.reshape(B, S, D)
